# Initial kernel scaffold; baseline (speedup 1.0000x reference)
import jax
import jax.numpy as jnp
from jax.experimental import pallas as pl


def kernel(element, x, embed_table):
    raise NotImplementedError("write your pallas kernel here")


# speedup vs baseline: 1.7316x; 1.7316x over previous
import functools

import jax
import jax.numpy as jnp
from jax import lax
from jax.experimental import pallas as pl
from jax.experimental.pallas import tpu as pltpu
from jax.experimental.pallas import tpu_sc as plsc

N = 100000
D = 128
DO = 256
C = 400
NCHUNK = N // C
NW = 32
CPW = -(-NCHUNK // NW)


def _body(element_hbm, x_hbm, table_hbm, out_hbm, idx_v, emb_v, x_v, sem):
    wid = lax.axis_index("s") * 2 + lax.axis_index("c")
    for j in range(CPW):
        cid = wid + j * NW

        @pl.when(cid < NCHUNK)
        def _():
            base = cid * C
            pltpu.sync_copy(element_hbm.at[pl.ds(base, C)], idx_v)
            pltpu.async_copy(table_hbm.at[idx_v], emb_v, sem).wait()
            pltpu.sync_copy(x_hbm.at[pl.ds(base, C), :], x_v)
            pltpu.sync_copy(emb_v, out_hbm.at[pl.ds(base, C), pl.ds(0, D)])
            pltpu.sync_copy(x_v, out_hbm.at[pl.ds(base, C), pl.ds(D, D)])


@jax.jit
def _sc_embed_concat(element, x, embed_table):
    mesh = plsc.VectorSubcoreMesh(core_axis_name="c", subcore_axis_name="s")
    return pl.kernel(
        _body,
        out_type=jax.ShapeDtypeStruct((N, DO), jnp.float32),
        mesh=mesh,
        scratch_types=[
            pltpu.VMEM((C,), jnp.int32),
            pltpu.VMEM((C, D), jnp.float32),
            pltpu.VMEM((C, D), jnp.float32),
            pltpu.SemaphoreType.DMA,
        ],
    )(element, x, embed_table)


def kernel(element, x, embed_table):
    return _sc_embed_concat(element.astype(jnp.int32), x, embed_table)
